# baseline (speedup 1.0000x reference)
import functools
import math

import jax
import jax.numpy as jnp
from jax import lax
from jax.experimental import pallas as pl
from jax.experimental.pallas import tpu as pltpu
from jax.experimental.pallas import tpu_sc as plsc

_MARGIN = 0.2
_P_TARGET = 0.1

_SC_ROWS = 8192
_NUM_CORES = 2
_NUM_SUBCORES = 16
_NW = _NUM_CORES * _NUM_SUBCORES
_CHUNK = 32

_RING_ROWS = 1024
_NBUF = 4

_LANES = 16
_NEG_SENTINEL = -1.0
_POS_INIT = 2.0


def _sc_part(dist, labels, c):
    rows_w = _SC_ROWS // _NW
    n_chunks = rows_w // _CHUNK
    n_full = c // _LANES
    tail_off = c - _LANES

    mesh = plsc.VectorSubcoreMesh(
        core_axis_name="c", subcore_axis_name="s",
        num_cores=_NUM_CORES, num_subcores=_NUM_SUBCORES)

    @functools.partial(
        pl.kernel,
        out_type=[
            jax.ShapeDtypeStruct((_NW, _LANES), jnp.float32),
            jax.ShapeDtypeStruct((_NW, _LANES), jnp.float32),
        ],
        mesh=mesh,
        scratch_types=[
            pltpu.VMEM((_CHUNK, c), jnp.float32),
            pltpu.VMEM((rows_w,), jnp.int32),
            pltpu.VMEM((_LANES,), jnp.float32),
            pltpu.VMEM((_LANES,), jnp.float32),
        ],
        compiler_params=pltpu.CompilerParams(needs_layout_passes=False),
    )
    def sc_k(dist_hbm, labels_hbm, pos_out, neg_out,
             chunk_v, labels_v, pos_v, neg_v):
        cid = lax.axis_index("c")
        sid = lax.axis_index("s")
        wid = sid * _NUM_CORES + cid
        base = wid * rows_w
        pltpu.sync_copy(labels_hbm.at[pl.ds(base, rows_w)], labels_v)

        pos0 = jnp.full((_LANES,), _POS_INIT, jnp.float32)
        neg0 = jnp.full((_LANES,), _NEG_SENTINEL, jnp.float32)

        def chunk_body(ci, accs):
            pos_acc, neg_acc = accs
            pltpu.sync_copy(
                dist_hbm.at[pl.ds(base + ci * _CHUNK, _CHUNK)], chunk_v)

            def g_body(g, p_acc):
                row_idx = lax.iota(jnp.int32, _LANES) + g * _LANES
                col_idx = labels_v[
                    pl.ds(pl.multiple_of(ci * _CHUNK + g * _LANES, _LANES),
                          _LANES)]
                p = plsc.load_gather(chunk_v, [row_idx, col_idx])
                plsc.store_scatter(
                    chunk_v, [row_idx, col_idx],
                    jnp.full((_LANES,), _NEG_SENTINEL, jnp.float32))
                return jnp.minimum(p_acc, p)

            pos_acc = lax.fori_loop(0, _CHUNK // _LANES, g_body, pos_acc)

            def r_body(r, n_acc):
                for o in range(n_full):
                    n_acc = jnp.maximum(
                        n_acc, chunk_v[r, pl.ds(o * _LANES, _LANES)])
                if tail_off % _LANES != 0:
                    n_acc = jnp.maximum(
                        n_acc, chunk_v[r, pl.ds(tail_off, _LANES)])
                return n_acc

            neg_acc = lax.fori_loop(0, _CHUNK, r_body, neg_acc)
            return pos_acc, neg_acc

        pos_acc, neg_acc = lax.fori_loop(
            0, n_chunks, chunk_body, (pos0, neg0))
        pos_v[...] = pos_acc
        neg_v[...] = neg_acc
        pltpu.sync_copy(pos_v, pos_out.at[wid])
        pltpu.sync_copy(neg_v, neg_out.at[wid])

    return sc_k(dist, labels)


def _tc_ring(dist, labels2, c, row_start, rows_total):
    n_blocks = rows_total // _RING_ROWS

    def body(dist_hbm, labels_hbm, out_ref, *scratch):
        bufs = scratch[:_NBUF]
        sems = scratch[_NBUF:2 * _NBUF]
        labels_v = scratch[2 * _NBUF]
        lsem = scratch[2 * _NBUF + 1]

        lcopy = pltpu.make_async_copy(
            labels_hbm.at[pl.ds(row_start, rows_total)], labels_v, lsem)
        lcopy.start()

        def dist_copy(j, idx):
            return pltpu.make_async_copy(
                dist_hbm.at[pl.ds(row_start + idx * _RING_ROWS, _RING_ROWS)],
                bufs[j], sems[j])

        for j in range(_NBUF):
            dist_copy(j, j).start()
        lcopy.wait()

        def loop(g, accs):
            pos, neg = accs
            for j in range(_NBUF):
                idx = g * _NBUF + j
                dist_copy(j, idx).wait()
                blk = bufs[j][...]
                lab = labels_v[pl.ds(idx * _RING_ROWS, _RING_ROWS), :]
                col = jax.lax.broadcasted_iota(jnp.int32, blk.shape, 1)
                is_pos = col == lab
                pos = jnp.minimum(
                    pos, jnp.min(jnp.where(is_pos, blk, jnp.inf)))
                neg = jnp.maximum(
                    neg, jnp.max(jnp.where(is_pos, -jnp.inf, blk)))
                nxt = idx + _NBUF

                @pl.when(nxt < n_blocks)
                def _():
                    dist_copy(j, nxt).start()
            return pos, neg

        pos, neg = lax.fori_loop(
            0, n_blocks // _NBUF, loop,
            (jnp.float32(jnp.inf), jnp.float32(-jnp.inf)))
        out_ref[0] = pos
        out_ref[1] = neg

    return pl.pallas_call(
        body,
        in_specs=[pl.BlockSpec(memory_space=pltpu.HBM),
                  pl.BlockSpec(memory_space=pltpu.HBM)],
        out_specs=pl.BlockSpec(memory_space=pltpu.SMEM),
        out_shape=jax.ShapeDtypeStruct((2,), jnp.float32),
        scratch_shapes=[pltpu.VMEM((_RING_ROWS, c), jnp.float32)] * _NBUF
        + [pltpu.SemaphoreType.DMA] * _NBUF
        + [pltpu.VMEM((rows_total, 1), jnp.int32),
           pltpu.SemaphoreType.DMA],
    )(dist, labels2)


@jax.jit
def kernel(dist, labels):
    b, c = dist.shape
    labels2 = labels.reshape(b, 1)

    tc_out = _tc_ring(dist, labels2, c, _SC_ROWS, b - _SC_ROWS)
    pos_min, neg_max = tc_out[0], tc_out[1]

    if _SC_ROWS:
        dist_b = lax.optimization_barrier(dist)
        sc_pos, sc_neg = _sc_part(
            lax.slice(dist_b, (0, 0), (_SC_ROWS, c)),
            lax.slice(labels, (0,), (_SC_ROWS,)), c)
        pos_min = jnp.minimum(pos_min, jnp.min(sc_pos))
        neg_max = jnp.maximum(neg_max, jnp.max(sc_neg))

    pos_theta = jnp.arccos(pos_min)
    neg_theta = jnp.arccos(neg_max)
    return _P_TARGET * jnp.maximum(pos_theta, _MARGIN) + (
        _P_TARGET - 1.0
    ) * jnp.minimum(neg_theta, 0.5 * math.pi - _MARGIN)


# speedup vs baseline: 1.6597x; 1.6597x over previous
import functools
import math

import jax
import jax.numpy as jnp
from jax import lax
from jax.experimental import pallas as pl
from jax.experimental.pallas import tpu as pltpu
from jax.experimental.pallas import tpu_sc as plsc

_MARGIN = 0.2
_P_TARGET = 0.1

_SC_ROWS = 0
_NUM_CORES = 2
_NUM_SUBCORES = 16
_NW = _NUM_CORES * _NUM_SUBCORES
_CHUNK = 32

_RING_ROWS = 1024
_NBUF = 4

_LANES = 16
_NEG_SENTINEL = -1.0
_POS_INIT = 2.0


def _sc_part(dist, labels, c):
    rows_w = _SC_ROWS // _NW
    n_chunks = rows_w // _CHUNK
    n_full = c // _LANES
    tail_off = c - _LANES

    mesh = plsc.VectorSubcoreMesh(
        core_axis_name="c", subcore_axis_name="s",
        num_cores=_NUM_CORES, num_subcores=_NUM_SUBCORES)

    @functools.partial(
        pl.kernel,
        out_type=[
            jax.ShapeDtypeStruct((_NW, _LANES), jnp.float32),
            jax.ShapeDtypeStruct((_NW, _LANES), jnp.float32),
        ],
        mesh=mesh,
        scratch_types=[
            pltpu.VMEM((_CHUNK, c), jnp.float32),
            pltpu.VMEM((rows_w,), jnp.int32),
            pltpu.VMEM((_LANES,), jnp.float32),
            pltpu.VMEM((_LANES,), jnp.float32),
        ],
        compiler_params=pltpu.CompilerParams(needs_layout_passes=False),
    )
    def sc_k(dist_hbm, labels_hbm, pos_out, neg_out,
             chunk_v, labels_v, pos_v, neg_v):
        cid = lax.axis_index("c")
        sid = lax.axis_index("s")
        wid = sid * _NUM_CORES + cid
        base = wid * rows_w
        pltpu.sync_copy(labels_hbm.at[pl.ds(base, rows_w)], labels_v)

        pos0 = jnp.full((_LANES,), _POS_INIT, jnp.float32)
        neg0 = jnp.full((_LANES,), _NEG_SENTINEL, jnp.float32)

        def chunk_body(ci, accs):
            pos_acc, neg_acc = accs
            pltpu.sync_copy(
                dist_hbm.at[pl.ds(base + ci * _CHUNK, _CHUNK)], chunk_v)

            def g_body(g, p_acc):
                row_idx = lax.iota(jnp.int32, _LANES) + g * _LANES
                col_idx = labels_v[
                    pl.ds(pl.multiple_of(ci * _CHUNK + g * _LANES, _LANES),
                          _LANES)]
                p = plsc.load_gather(chunk_v, [row_idx, col_idx])
                plsc.store_scatter(
                    chunk_v, [row_idx, col_idx],
                    jnp.full((_LANES,), _NEG_SENTINEL, jnp.float32))
                return jnp.minimum(p_acc, p)

            pos_acc = lax.fori_loop(0, _CHUNK // _LANES, g_body, pos_acc)

            def r_body(r, n_acc):
                for o in range(n_full):
                    n_acc = jnp.maximum(
                        n_acc, chunk_v[r, pl.ds(o * _LANES, _LANES)])
                if tail_off % _LANES != 0:
                    n_acc = jnp.maximum(
                        n_acc, chunk_v[r, pl.ds(tail_off, _LANES)])
                return n_acc

            neg_acc = lax.fori_loop(0, _CHUNK, r_body, neg_acc)
            return pos_acc, neg_acc

        pos_acc, neg_acc = lax.fori_loop(
            0, n_chunks, chunk_body, (pos0, neg0))
        pos_v[...] = pos_acc
        neg_v[...] = neg_acc
        pltpu.sync_copy(pos_v, pos_out.at[wid])
        pltpu.sync_copy(neg_v, neg_out.at[wid])

    return sc_k(dist, labels)


def _tc_ring(dist, labels2, c, row_start, rows_total):
    n_blocks = rows_total // _RING_ROWS

    def body(dist_hbm, labels_hbm, out_ref, *scratch):
        bufs = scratch[:_NBUF]
        sems = scratch[_NBUF:2 * _NBUF]
        labels_v = scratch[2 * _NBUF]
        lsem = scratch[2 * _NBUF + 1]

        lcopy = pltpu.make_async_copy(
            labels_hbm.at[pl.ds(row_start, rows_total)], labels_v, lsem)
        lcopy.start()

        def dist_copy(j, idx):
            return pltpu.make_async_copy(
                dist_hbm.at[pl.ds(row_start + idx * _RING_ROWS, _RING_ROWS)],
                bufs[j], sems[j])

        for j in range(_NBUF):
            dist_copy(j, j).start()
        lcopy.wait()

        def loop(g, accs):
            pos, neg = accs
            for j in range(_NBUF):
                idx = g * _NBUF + j
                dist_copy(j, idx).wait()
                blk = bufs[j][...]
                lab = labels_v[pl.ds(idx * _RING_ROWS, _RING_ROWS), :]
                col = jax.lax.broadcasted_iota(jnp.int32, blk.shape, 1)
                masked = blk + jnp.where(col == lab, -2.0, 0.0)
                pos = jnp.minimum(pos, jnp.min(masked))
                neg = jnp.maximum(neg, jnp.max(masked))
                nxt = idx + _NBUF

                @pl.when(nxt < n_blocks)
                def _():
                    dist_copy(j, nxt).start()
            return pos, neg

        pos, neg = lax.fori_loop(
            0, n_blocks // _NBUF, loop,
            (jnp.float32(jnp.inf), jnp.float32(-jnp.inf)))
        out_ref[0] = pos + 2.0
        out_ref[1] = neg

    return pl.pallas_call(
        body,
        in_specs=[pl.BlockSpec(memory_space=pltpu.HBM),
                  pl.BlockSpec(memory_space=pltpu.HBM)],
        out_specs=pl.BlockSpec(memory_space=pltpu.SMEM),
        out_shape=jax.ShapeDtypeStruct((2,), jnp.float32),
        scratch_shapes=[pltpu.VMEM((_RING_ROWS, c), jnp.float32)] * _NBUF
        + [pltpu.SemaphoreType.DMA] * _NBUF
        + [pltpu.VMEM((rows_total, 1), jnp.int32),
           pltpu.SemaphoreType.DMA],
    )(dist, labels2)


@jax.jit
def kernel(dist, labels):
    b, c = dist.shape
    labels2 = labels.reshape(b, 1)

    tc_out = _tc_ring(dist, labels2, c, _SC_ROWS, b - _SC_ROWS)
    pos_min, neg_max = tc_out[0], tc_out[1]

    if _SC_ROWS:
        dist_b = lax.optimization_barrier(dist)
        sc_pos, sc_neg = _sc_part(
            lax.slice(dist_b, (0, 0), (_SC_ROWS, c)),
            lax.slice(labels, (0,), (_SC_ROWS,)), c)
        pos_min = jnp.minimum(pos_min, jnp.min(sc_pos))
        neg_max = jnp.maximum(neg_max, jnp.max(sc_neg))

    pos_theta = jnp.arccos(pos_min)
    neg_theta = jnp.arccos(neg_max)
    return _P_TARGET * jnp.maximum(pos_theta, _MARGIN) + (
        _P_TARGET - 1.0
    ) * jnp.minimum(neg_theta, 0.5 * math.pi - _MARGIN)
